# baseline (speedup 1.0000x reference)
import functools

import jax
import jax.numpy as jnp
from jax import lax
from jax.experimental import pallas as pl
from jax.experimental.pallas import tpu as pltpu
from jax.experimental.pallas import tpu_sc as plsc

N_NODES = 10000
N_EDGES = 320000
D = 128

NW = 32
NPW = 320
NPAD = NW * NPW

ECHUNK = 1600
NCHUNK = N_EDGES // ECHUNK
BATCH = 128
BUFCAP = 160
SRC_BITS = 15
SRC_MASK = (1 << SRC_BITS) - 1
NEG_INF = float("-inf")


def _mm_body(x_ref, w_ref, b_ref, p_ref, q_ref):
    xb = x_ref[:]
    w1 = w_ref[0:D, :]
    w2 = w_ref[D:2 * D, :]
    q = jnp.dot(xb, w2, preferred_element_type=jnp.float32)
    p = jnp.dot(xb, w1, preferred_element_type=jnp.float32) - q + b_ref[:]
    p_ref[:] = p
    q_ref[:] = q


def _matmul_pq(xp, W, b2d):
    blk = 1024
    grid = NPAD // blk
    return pl.pallas_call(
        _mm_body,
        grid=(grid,),
        in_specs=[
            pl.BlockSpec((blk, D), lambda i: (i, 0)),
            pl.BlockSpec((2 * D, D), lambda i: (0, 0)),
            pl.BlockSpec((1, D), lambda i: (0, 0)),
        ],
        out_specs=[
            pl.BlockSpec((blk, D), lambda i: (i, 0)),
            pl.BlockSpec((blk, D), lambda i: (i, 0)),
        ],
        out_shape=[
            jax.ShapeDtypeStruct((NPAD, D), jnp.float32),
            jax.ShapeDtypeStruct((NPAD, D), jnp.float32),
        ],
    )(xp, W, b2d)


@functools.partial(
    pl.kernel,
    mesh=plsc.VectorSubcoreMesh(core_axis_name="c", subcore_axis_name="s"),
    compiler_params=pltpu.CompilerParams(needs_layout_passes=False),
    out_type=jax.ShapeDtypeStruct((NPAD, D), jnp.float32),
    scratch_types=[
        pltpu.VMEM((NPW + 8, D), jnp.float32),
        pltpu.VMEM((2 * BATCH, D), jnp.float32),
        pltpu.VMEM((ECHUNK,), jnp.int32),
        pltpu.VMEM((ECHUNK,), jnp.int32),
        pltpu.VMEM((BUFCAP,), jnp.int32),
        pltpu.VMEM((2 * BUFCAP,), jnp.int32),
        pltpu.VMEM((2 * BUFCAP,), jnp.int32),
        pltpu.VMEM((64, D), jnp.float32),
        pltpu.SemaphoreType.DMA,
        pltpu.SemaphoreType.DMA,
        pltpu.SemaphoreType.DMA,
        pltpu.SemaphoreType.DMA,
        pltpu.SemaphoreType.DMA,
    ],
)
def _segmax_kernel(p_hbm, q_hbm, edges_hbm, out_hbm,
                   acc, rows, ebuf0, ebuf1,
                   bpak, bsrc, bldst, pbuf,
                   semg0, semg1, semt, seme0, seme1):
    wid = lax.axis_index("s") * 2 + lax.axis_index("c")
    lo = wid * NPW
    plo = lo * (1 << SRC_BITS)
    phi = plo + NPW * (1 << SRC_BITS)

    neg = jnp.full((16,), NEG_INF, jnp.float32)

    def init_body(i, _):
        def init_k(k, _2):
            acc[i, pl.ds(k * 16, 16)] = neg
            return 0

        lax.fori_loop(0, D // 16, init_k, 0, unroll=True)
        return 0

    lax.fori_loop(0, NPW + 8, init_body, 0)

    def unpack(slot, nwords):
        base = slot * BUFCAP
        for k in range(nwords):
            v = bpak[pl.ds(k * 16, 16)]
            bsrc[pl.ds(base + k * 16, 16)] = jnp.bitwise_and(v, SRC_MASK)
            bldst[pl.ds(base + k * 16, 16)] = jnp.right_shift(v, SRC_BITS) - lo

    def fold16(slot, eb):
        dvec16 = bldst[pl.ds(slot * BUFCAP + eb * 16, 16)]
        rbase = slot * BATCH + eb * 16
        for e in range(16):
            d = dvec16[e]

            def k_step(k, _2, e=e, d=d):
                g = rows[rbase + e, pl.ds(k * 16, 16)]
                a = acc[d, pl.ds(k * 16, 16)]
                acc[d, pl.ds(k * 16, 16)] = jnp.maximum(a, g)
                return 0

            lax.fori_loop(0, D // 16, k_step, 0, unroll=True)

    def fold_batch(slot):
        def batch_body(eb, _):
            fold16(slot, eb)
            return 0

        lax.fori_loop(0, BATCH // 16, batch_body, 0)

    def issue_gather(p):
        pltpu.async_copy(
            q_hbm.at[bsrc.at[pl.ds(p * BUFCAP, BATCH)]],
            rows.at[pl.ds(p * BATCH, BATCH)],
            (semg0, semg1)[p])

    def wait_gather(p):
        pltpu.make_async_copy(
            q_hbm.at[bsrc.at[pl.ds(p * BUFCAP, BATCH)]],
            rows.at[pl.ds(p * BATCH, BATCH)],
            (semg0, semg1)[p]).wait()

    def drain128(dc):
        p = jnp.bitwise_and(dc, 1)

        @pl.when(dc > 0)
        def _():
            @pl.when(p == 0)
            def _():
                wait_gather(1)
                fold_batch(1)

            @pl.when(p == 1)
            def _():
                wait_gather(0)
                fold_batch(0)

        @pl.when(p == 0)
        def _():
            unpack0()
            issue_gather(0)

        @pl.when(p == 1)
        def _():
            unpack1()
            issue_gather(1)

        bpak[pl.ds(0, 16)] = bpak[pl.ds(BATCH, 16)]

    def unpack0():
        unpack(0, BATCH // 16)

    def unpack1():
        unpack(1, BATCH // 16)

    def make_scan16(eb):
        def scan16(j, carry):
            n, dc = carry
            pkv = eb[pl.ds(j * 16, 16)]
            m = (pkv >= plo) & (pkv < phi)
            _, cpak, _ = plsc.sort_key_val(pkv, pkv, mask=m)
            bpak[pl.ds(n, 16)] = cpak
            cnt = plsc.all_reduce_population_count(m)[0]
            n = n + cnt
            full = n >= BATCH

            @pl.when(full)
            def _():
                drain128(dc)

            return (jnp.where(full, n - BATCH, n),
                    jnp.where(full, dc + 1, dc))

        return scan16

    sem_e = (seme0, seme1)
    ebufs = (ebuf0, ebuf1)

    def stage(ci, b):
        pltpu.async_copy(edges_hbm.at[pl.ds(ci * ECHUNK, ECHUNK)],
                         ebufs[b], sem_e[b])

    def wait_stage(b):
        pltpu.make_async_copy(edges_hbm.at[pl.ds(0, ECHUNK)],
                              ebufs[b], sem_e[b]).wait()

    stage(0, 0)
    stage(1, 1)

    def chunk_pair(h, carry):
        for b in range(2):
            ci = h * 2 + b
            wait_stage(b)
            carry = lax.fori_loop(0, ECHUNK // 16, make_scan16(ebufs[b]),
                                  carry, unroll=2)

            @pl.when(ci + 2 < NCHUNK)
            def _(ci=ci, b=b):
                stage(ci + 2, b)

        return carry

    n, dc = lax.fori_loop(0, NCHUNK // 2, chunk_pair,
                          (jnp.int32(0), jnp.int32(0)))

    p = jnp.bitwise_and(dc, 1)

    @pl.when((dc > 0) & (p == 1))
    def _():
        wait_gather(0)
        fold_batch(0)

    @pl.when((dc > 0) & (p == 0))
    def _():
        wait_gather(1)
        fold_batch(1)

    bpak[pl.ds(n, 16)] = (jnp.zeros((16,), jnp.int32)
                          + (lo + NPW) * (1 << SRC_BITS))
    unpack(0, BUFCAP // 16)
    nb = (n + 15) // 16

    def tail_batch(t, _):
        pltpu.async_copy(q_hbm.at[bsrc.at[pl.ds(t * 16, 16)]],
                         rows.at[pl.ds(t * 16, 16)], semt).wait()
        fold16(0, t)
        return 0

    lax.fori_loop(0, nb, tail_batch, 0)

    for g in range(NPW // 64):
        pltpu.sync_copy(p_hbm.at[pl.ds(lo + g * 64, 64)], pbuf)

        def row_body(r, _):
            def k_step(k, _2):
                pv = pbuf[r, pl.ds(k * 16, 16)]
                a = acc[g * 64 + r, pl.ds(k * 16, 16)]
                pbuf[r, pl.ds(k * 16, 16)] = jnp.maximum(pv + a, 0.0)
                return 0

            lax.fori_loop(0, D // 16, k_step, 0, unroll=True)
            return 0

        lax.fori_loop(0, 64, row_body, 0)
        pltpu.sync_copy(pbuf, out_hbm.at[pl.ds(lo + g * 64, 64)])


def kernel(x, edge_index, W, b):
    src = edge_index[0].astype(jnp.int32)
    dst = edge_index[1].astype(jnp.int32)
    edges = dst * (1 << SRC_BITS) + src
    xp = jnp.pad(x, ((0, NPAD - N_NODES), (0, 0)))
    P, Q = _matmul_pq(xp, W, b.reshape(1, D))
    out = _segmax_kernel(P, Q, edges)
    return out[:N_NODES]


# speedup vs baseline: 1.3276x; 1.3276x over previous
import functools

import jax
import jax.numpy as jnp
from jax import lax
from jax.experimental import pallas as pl
from jax.experimental.pallas import tpu as pltpu
from jax.experimental.pallas import tpu_sc as plsc

N_NODES = 10000
N_EDGES = 320000
D = 128

NW = 32
NPW = 320
NPAD = NW * NPW

ECHUNK = 1600
NCHUNK = N_EDGES // ECHUNK
BATCH = 128
BUFCAP = 160
SRC_BITS = 15
SRC_MASK = (1 << SRC_BITS) - 1
NEG_INF = float("-inf")


def _mm_body(x_ref, w_ref, b_ref, p_ref, q_ref):
    xb = x_ref[:]
    w1 = w_ref[0:D, :]
    w2 = w_ref[D:2 * D, :]
    q = jnp.dot(xb, w2, preferred_element_type=jnp.float32)
    p = jnp.dot(xb, w1, preferred_element_type=jnp.float32) - q + b_ref[:]
    p_ref[:] = p
    q_ref[:] = q


def _matmul_pq(xp, W, b2d):
    blk = 1024
    grid = NPAD // blk
    return pl.pallas_call(
        _mm_body,
        grid=(grid,),
        in_specs=[
            pl.BlockSpec((blk, D), lambda i: (i, 0)),
            pl.BlockSpec((2 * D, D), lambda i: (0, 0)),
            pl.BlockSpec((1, D), lambda i: (0, 0)),
        ],
        out_specs=[
            pl.BlockSpec((blk, D), lambda i: (i, 0)),
            pl.BlockSpec((blk, D), lambda i: (i, 0)),
        ],
        out_shape=[
            jax.ShapeDtypeStruct((NPAD, D), jnp.float32),
            jax.ShapeDtypeStruct((NPAD, D), jnp.float32),
        ],
    )(xp, W, b2d)


@functools.partial(
    pl.kernel,
    mesh=plsc.VectorSubcoreMesh(core_axis_name="c", subcore_axis_name="s"),
    compiler_params=pltpu.CompilerParams(needs_layout_passes=False),
    out_type=jax.ShapeDtypeStruct((NPAD, D), jnp.float32),
    scratch_types=[
        pltpu.VMEM((NPW + 8, D), jnp.float32),
        pltpu.VMEM((2 * BATCH, D), jnp.float32),
        pltpu.VMEM((ECHUNK,), jnp.int32),
        pltpu.VMEM((ECHUNK,), jnp.int32),
        pltpu.VMEM((BUFCAP,), jnp.int32),
        pltpu.VMEM((2 * BUFCAP,), jnp.int32),
        pltpu.VMEM((2 * BUFCAP,), jnp.int32),
        pltpu.VMEM((64, D), jnp.float32),
        pltpu.SemaphoreType.DMA,
        pltpu.SemaphoreType.DMA,
        pltpu.SemaphoreType.DMA,
        pltpu.SemaphoreType.DMA,
        pltpu.SemaphoreType.DMA,
    ],
)
def _segmax_kernel(p_hbm, q_hbm, edges_hbm, out_hbm,
                   acc, rows, ebuf0, ebuf1,
                   bpak, bsrc, bldst, pbuf,
                   semg0, semg1, semt, seme0, seme1):
    wid = lax.axis_index("s") * 2 + lax.axis_index("c")
    lo = wid * NPW
    plo = lo * (1 << SRC_BITS)
    phi = plo + NPW * (1 << SRC_BITS)

    neg = jnp.full((16,), NEG_INF, jnp.float32)

    def init_body(i, _):
        def init_k(k, _2):
            acc[i, pl.ds(k * 16, 16)] = neg
            return 0

        lax.fori_loop(0, D // 16, init_k, 0, unroll=True)
        return 0

    lax.fori_loop(0, NPW + 8, init_body, 0)

    def unpack(slot, nwords):
        base = slot * BUFCAP
        for k in range(nwords):
            v = bpak[pl.ds(k * 16, 16)]
            bsrc[pl.ds(base + k * 16, 16)] = jnp.bitwise_and(v, SRC_MASK)
            bldst[pl.ds(base + k * 16, 16)] = jnp.right_shift(v, SRC_BITS) - lo

    def fold16(slot, eb):
        dvec16 = bldst[pl.ds(slot * BUFCAP + eb * 16, 16)]
        rbase = slot * BATCH + eb * 16
        for e in range(16):
            d = dvec16[e]

            def k_step(k, _2, e=e, d=d):
                g = rows[rbase + e, pl.ds(k * 16, 16)]
                a = acc[d, pl.ds(k * 16, 16)]
                acc[d, pl.ds(k * 16, 16)] = jnp.maximum(a, g)
                return 0

            lax.fori_loop(0, D // 16, k_step, 0, unroll=True)

    def fold_batch(slot):
        def batch_body(eb, _):
            fold16(slot, eb)
            return 0

        lax.fori_loop(0, BATCH // 16, batch_body, 0)

    def issue_gather(p):
        pltpu.async_copy(
            q_hbm.at[bsrc.at[pl.ds(p * BUFCAP, BATCH)]],
            rows.at[pl.ds(p * BATCH, BATCH)],
            (semg0, semg1)[p])

    def wait_gather(p):
        pltpu.make_async_copy(
            q_hbm.at[bsrc.at[pl.ds(p * BUFCAP, BATCH)]],
            rows.at[pl.ds(p * BATCH, BATCH)],
            (semg0, semg1)[p]).wait()

    def drain128(dc):
        p = jnp.bitwise_and(dc, 1)

        @pl.when(dc > 0)
        def _():
            @pl.when(p == 0)
            def _():
                wait_gather(1)
                fold_batch(1)

            @pl.when(p == 1)
            def _():
                wait_gather(0)
                fold_batch(0)

        @pl.when(p == 0)
        def _():
            unpack0()
            issue_gather(0)

        @pl.when(p == 1)
        def _():
            unpack1()
            issue_gather(1)

        bpak[pl.ds(0, 16)] = bpak[pl.ds(BATCH, 16)]
        bpak[pl.ds(16, 16)] = bpak[pl.ds(BATCH + 16, 16)]

    def unpack0():
        unpack(0, BATCH // 16)

    def unpack1():
        unpack(1, BATCH // 16)

    def make_scan32(eb):
        def scan32(j, carry):
            n, dc = carry
            pkv0 = eb[pl.ds(j * 32, 16)]
            pkv1 = eb[pl.ds(j * 32 + 16, 16)]
            m0 = (pkv0 >= plo) & (pkv0 < phi)
            m1 = (pkv1 >= plo) & (pkv1 < phi)
            _, cpak0, _ = plsc.sort_key_val(pkv0, pkv0, mask=m0)
            _, cpak1, _ = plsc.sort_key_val(pkv1, pkv1, mask=m1)
            cnt0 = plsc.all_reduce_population_count(m0)[0]
            cnt1 = plsc.all_reduce_population_count(m1)[0]
            bpak[pl.ds(n, 16)] = cpak0
            bpak[pl.ds(n + cnt0, 16)] = cpak1
            n = n + cnt0 + cnt1
            full = n >= BATCH

            @pl.when(full)
            def _():
                drain128(dc)

            return (jnp.where(full, n - BATCH, n),
                    jnp.where(full, dc + 1, dc))

        return scan32

    sem_e = (seme0, seme1)
    ebufs = (ebuf0, ebuf1)

    def stage(ci, b):
        pltpu.async_copy(edges_hbm.at[pl.ds(ci * ECHUNK, ECHUNK)],
                         ebufs[b], sem_e[b])

    def wait_stage(b):
        pltpu.make_async_copy(edges_hbm.at[pl.ds(0, ECHUNK)],
                              ebufs[b], sem_e[b]).wait()

    stage(0, 0)
    stage(1, 1)

    def chunk_pair(h, carry):
        for b in range(2):
            ci = h * 2 + b
            wait_stage(b)
            carry = lax.fori_loop(0, ECHUNK // 32, make_scan32(ebufs[b]),
                                  carry, unroll=2)

            @pl.when(ci + 2 < NCHUNK)
            def _(ci=ci, b=b):
                stage(ci + 2, b)

        return carry

    n, dc = lax.fori_loop(0, NCHUNK // 2, chunk_pair,
                          (jnp.int32(0), jnp.int32(0)))

    p = jnp.bitwise_and(dc, 1)

    @pl.when((dc > 0) & (p == 1))
    def _():
        wait_gather(0)
        fold_batch(0)

    @pl.when((dc > 0) & (p == 0))
    def _():
        wait_gather(1)
        fold_batch(1)

    bpak[pl.ds(n, 16)] = (jnp.zeros((16,), jnp.int32)
                          + (lo + NPW) * (1 << SRC_BITS))
    unpack(0, BUFCAP // 16)
    nb = (n + 15) // 16

    def tail_batch(t, _):
        pltpu.async_copy(q_hbm.at[bsrc.at[pl.ds(t * 16, 16)]],
                         rows.at[pl.ds(t * 16, 16)], semt).wait()
        fold16(0, t)
        return 0

    lax.fori_loop(0, nb, tail_batch, 0)

    for g in range(NPW // 64):
        pltpu.sync_copy(p_hbm.at[pl.ds(lo + g * 64, 64)], pbuf)

        def row_body(r, _):
            def k_step(k, _2):
                pv = pbuf[r, pl.ds(k * 16, 16)]
                a = acc[g * 64 + r, pl.ds(k * 16, 16)]
                pbuf[r, pl.ds(k * 16, 16)] = jnp.maximum(pv + a, 0.0)
                return 0

            lax.fori_loop(0, D // 16, k_step, 0, unroll=True)
            return 0

        lax.fori_loop(0, 64, row_body, 0)
        pltpu.sync_copy(pbuf, out_hbm.at[pl.ds(lo + g * 64, 64)])


def kernel(x, edge_index, W, b):
    src = edge_index[0].astype(jnp.int32)
    dst = edge_index[1].astype(jnp.int32)
    edges = dst * (1 << SRC_BITS) + src
    xp = jnp.pad(x, ((0, NPAD - N_NODES), (0, 0)))
    P, Q = _matmul_pq(xp, W, b.reshape(1, D))
    out = _segmax_kernel(P, Q, edges)
    return out[:N_NODES]
